# baseline (speedup 1.0000x reference)
import functools

import jax
import jax.numpy as jnp
from jax import lax
from jax.experimental import pallas as pl
from jax.experimental.pallas import tpu as pltpu
from jax.experimental.pallas import tpu_sc as plsc

NUM_CLASSES = 1000000
DIM = 16
BATCH = 16384

_INFO = plsc.get_sparse_core_info()
_NC, _NS, _L = _INFO.num_cores, _INFO.num_subcores, _INFO.num_lanes
_NW = _NC * _NS
_BPW = BATCH // _NW
_CHUNK = 128
_NCHUNK = _BPW // _CHUNK


def _sc_gather(y_hbm, emb_hbm, out_hbm, idx_raw, idx_m, rows, sem):
    wid = lax.axis_index("s") * _NC + lax.axis_index("c")
    base = wid * _BPW
    pltpu.sync_copy(y_hbm.at[pl.ds(base, _BPW)], idx_raw)
    null_v = jnp.full((_L,), NUM_CLASSES, dtype=jnp.int32)
    for j in range(_NCHUNK):
        for k in range(_CHUNK // _L):
            v = idx_raw[pl.ds(j * _CHUNK + k * _L, _L)]
            idx_m[j, pl.ds(k * _L, _L)] = jnp.where(v < 0, null_v, v)
    copies = [
        pltpu.async_copy(
            emb_hbm.at[idx_m.at[j]],
            rows.at[pl.ds(j * _CHUNK, _CHUNK)],
            sem,
        )
        for j in range(_NCHUNK)
    ]
    for c in copies:
        c.wait()
    pltpu.sync_copy(rows, out_hbm.at[pl.ds(base, _BPW)])


@jax.jit
def kernel(y, emb):
    mesh = plsc.VectorSubcoreMesh(core_axis_name="c", subcore_axis_name="s")
    run = pl.kernel(
        _sc_gather,
        mesh=mesh,
        out_type=jax.ShapeDtypeStruct((BATCH, DIM), jnp.float32),
        scratch_types=[
            pltpu.VMEM((_BPW,), jnp.int32),
            pltpu.VMEM((_NCHUNK, _CHUNK), jnp.int32),
            pltpu.VMEM((_BPW, DIM), jnp.float32),
            pltpu.SemaphoreType.DMA,
        ],
        compiler_params=pltpu.CompilerParams(use_tc_tiling_on_sc=False),
    )
    return run(y.astype(jnp.int32), emb)
